# baseline (speedup 1.0000x reference)
import functools

import jax
import jax.numpy as jnp
import numpy as np
from jax import lax
from jax.experimental import pallas as pl
from jax.experimental.pallas import tpu as pltpu
from jax.experimental.pallas import tpu_sc as plsc

_VOCAB = 1000000
_EMBED = 128
_B = 16384
_ND = 13
_NS = 26
_NP = 32

_NC = 2
_NSUB = 16
_NW = _NC * _NSUB

_NSLICE = 4
_BS = _B // _NSLICE
_ROWS_S = _BS * _NP
_CHUNK = 128
_CH_S = _ROWS_S // (_NW * _CHUNK)


def _sc_gather(emb, idx3):
    mesh = plsc.VectorSubcoreMesh(core_axis_name="c", subcore_axis_name="s")
    half = _ROWS_S // _CHUNK // 2

    @functools.partial(
        pl.kernel,
        out_type=(jax.ShapeDtypeStruct((half, _CHUNK, _EMBED), jnp.float32),
                  jax.ShapeDtypeStruct((half, _CHUNK, _EMBED), jnp.float32)),
        mesh=mesh,
        scratch_types=[
            pltpu.VMEM((_CH_S, _CHUNK), jnp.int32),
            pltpu.VMEM((_CHUNK, _EMBED), jnp.float32),
            pltpu.VMEM((_CHUNK, _EMBED), jnp.float32),
            pltpu.VMEM((_CHUNK, _EMBED), jnp.float32),
            pltpu.VMEM((_CHUNK, _EMBED), jnp.float32),
            pltpu.SemaphoreType.DMA,
            pltpu.SemaphoreType.DMA,
            pltpu.SemaphoreType.DMA,
            pltpu.SemaphoreType.DMA,
        ],
    )
    def body(emb_hbm, idx_hbm, out0_hbm, out1_hbm, idx_v,
             buf0, buf1, buf2, buf3, sem0, sem1, sem2, sem3):
        bufs = (buf0, buf1, buf2, buf3)
        sems = (sem0, sem1, sem2, sem3)
        wid = lax.axis_index("s") * _NC + lax.axis_index("c")
        pltpu.sync_copy(idx_hbm.at[wid], idx_v)
        for u in range(3):
            pltpu.make_async_copy(emb_hbm.at[idx_v.at[u]], bufs[u], sems[u]).start()

        def make_loop(out_hbm, cbase):
            def step(t, carry):
                j0 = t * 4
                for u in range(4):
                    j = j0 + u
                    nb = bufs[(u + 3) % 4]
                    ns = sems[(u + 3) % 4]

                    @pl.when(j + 3 < _CH_S)
                    def _():
                        pltpu.make_async_copy(emb_hbm.at[idx_v.at[j + 3]], nb, ns).start()

                    pltpu.make_async_copy(emb_hbm.at[idx_v.at[j]], bufs[u], sems[u]).wait()
                    pltpu.sync_copy(bufs[u], out_hbm.at[cbase + j])
                return carry
            return step

        @pl.when(wid < _NW // 2)
        def _():
            lax.fori_loop(0, _CH_S // 4, make_loop(out0_hbm, wid * _CH_S), 0)

        @pl.when(wid >= _NW // 2)
        def _():
            lax.fori_loop(0, _CH_S // 4, make_loop(out1_hbm, (wid - _NW // 2) * _CH_S), 0)

    return body(emb, idx3)


_BB = 512


def _half_interf(g, h2h):
    hb = _BB // 2
    g3 = g[...].reshape(hb * _NP, _EMBED).reshape(hb, _NP, _EMBED)
    slot = lax.broadcasted_iota(jnp.int32, (hb, _NP, _EMBED), 1)
    comb = jnp.where(slot == _NS, h2h.reshape(hb, 1, _EMBED), g3)
    inter = lax.dot_general(comb, comb, (((2,), (2,)), ((0,), (0,))),
                            preferred_element_type=jnp.float32)
    return inter.reshape(hb, _NP * _NP)


def _tc_body(dense, g0, g1, w0, b0, w1, b1, w2, b2,
             wh, wint, tb0, wt1, tb1, wt2, tb2, wt3, tb3, wt4, tb4, out):
    f32 = jnp.float32
    h = jnp.maximum(jnp.dot(dense[...], w0[...], preferred_element_type=f32) + b0[...], 0.0)
    h = jnp.maximum(jnp.dot(h, w1[...], preferred_element_type=f32) + b1[...], 0.0)
    h2 = jnp.maximum(jnp.dot(h, w2[...], preferred_element_type=f32) + b2[...], 0.0)
    interf = jnp.concatenate(
        [_half_interf(g0, h2[:_BB // 2]), _half_interf(g1, h2[_BB // 2:])], axis=0)
    t = jnp.dot(h2, wh[...], preferred_element_type=f32)
    t = t + jnp.dot(interf, wint[...], preferred_element_type=f32)
    t = jnp.maximum(t + tb0[...], 0.0)
    t = jnp.maximum(jnp.dot(t, wt1[...], preferred_element_type=f32) + tb1[...], 0.0)
    t = jnp.maximum(jnp.dot(t, wt2[...], preferred_element_type=f32) + tb2[...], 0.0)
    t = jnp.maximum(jnp.dot(t, wt3[...], preferred_element_type=f32) + tb3[...], 0.0)
    out[...] = jnp.dot(t, wt4[...], preferred_element_type=f32) + tb4[...]


def _tc_fused(dense, g0, g1, weights):
    grid = (_BS // _BB,)
    nch2 = _BB * _NP // _CHUNK // 2

    def blk(shape):
        return pl.BlockSpec(shape, lambda i: (i, 0))

    def rep(shape):
        return pl.BlockSpec(shape, lambda i: (0,) * len(shape))

    gspec = pl.BlockSpec((nch2, _CHUNK, _EMBED), lambda i: (i, 0, 0))
    in_specs = [blk((_BB, _ND)), gspec, gspec]
    for w in weights:
        in_specs.append(rep(w.shape))
    return pl.pallas_call(
        _tc_body,
        grid=grid,
        in_specs=in_specs,
        out_specs=blk((_BB, 1)),
        out_shape=jax.ShapeDtypeStruct((_BS, 1), jnp.float32),
    )(dense, g0, g1, *weights)


def kernel(x, emb, bw0, bb0, bw1, bb1, bw2, bb2,
           tw0, tb0, tw1, tb1, tw2, tb2, tw3, tb3, tw4, tb4):
    dense = x[:, :_ND]
    idx_all = x[:, _ND:].astype(jnp.int32) % _VOCAB
    npd = _NP - _NS
    dummy = (jnp.arange(_B, dtype=jnp.int32)[:, None] * npd
             + jnp.arange(npd, dtype=jnp.int32)[None, :]) % _VOCAB
    idx_pad = jnp.concatenate([idx_all, dummy], axis=1)

    iu, ku = np.triu_indices(_NS + 1)
    remap = np.concatenate([[26], np.arange(_NS)])
    ri = remap[iu]
    rk = remap[ku]
    lo = np.minimum(ri, rk)
    hi = np.maximum(ri, rk)
    rows = jnp.asarray(lo * _NP + hi, dtype=jnp.int32)
    wint = jnp.zeros((_NP * _NP, 1024), jnp.float32).at[rows].set(tw0[:, _EMBED:].T)

    weights = (
        bw0.T, bb0.reshape(1, -1), bw1.T, bb1.reshape(1, -1), bw2.T, bb2.reshape(1, -1),
        tw0[:, :_EMBED].T, wint, tb0.reshape(1, -1),
        tw1.T, tb1.reshape(1, -1), tw2.T, tb2.reshape(1, -1),
        tw3.T, tb3.reshape(1, -1), tw4.T, tb4.reshape(1, -1),
    )

    nchunks = _ROWS_S // _CHUNK
    cpb = _BB * _NP // _CHUNK
    hc = cpb // 2
    p = np.arange(nchunks // 2)
    c0 = (p // hc) * cpb + p % hc
    order = np.concatenate([c0, c0 + hc])

    parts = []
    for s in range(_NSLICE):
        idx_rows = idx_pad[s * _BS:(s + 1) * _BS].reshape(nchunks, _CHUNK)
        idx_s = idx_rows[order].reshape(_NW, _CH_S, _CHUNK)
        g0, g1 = _sc_gather(emb, idx_s)
        parts.append(_tc_fused(dense[s * _BS:(s + 1) * _BS], g0, g1, weights))
    return jnp.concatenate(parts, axis=0)


# speedup vs baseline: 1.0405x; 1.0405x over previous
import functools

import jax
import jax.numpy as jnp
import numpy as np
from jax import lax
from jax.experimental import pallas as pl
from jax.experimental.pallas import tpu as pltpu
from jax.experimental.pallas import tpu_sc as plsc

_VOCAB = 1000000
_EMBED = 128
_B = 16384
_ND = 13
_NS = 26
_NP = 32

_NC = 2
_NSUB = 16
_NW = _NC * _NSUB

_NSLICE = 4
_BS = _B // _NSLICE
_ROWS_S = _BS * _NP
_CHUNK = 128
_CH_S = _ROWS_S // (_NW * _CHUNK)


def _sc_gather(emb, idx3):
    mesh = plsc.VectorSubcoreMesh(core_axis_name="c", subcore_axis_name="s")

    @functools.partial(
        pl.kernel,
        out_type=jax.ShapeDtypeStruct((_ROWS_S // _CHUNK, _CHUNK, _EMBED), jnp.float32),
        mesh=mesh,
        scratch_types=[
            pltpu.VMEM((_CH_S, _CHUNK), jnp.int32),
            pltpu.VMEM((_CHUNK, _EMBED), jnp.float32),
            pltpu.VMEM((_CHUNK, _EMBED), jnp.float32),
            pltpu.VMEM((_CHUNK, _EMBED), jnp.float32),
            pltpu.VMEM((_CHUNK, _EMBED), jnp.float32),
            pltpu.SemaphoreType.DMA,
            pltpu.SemaphoreType.DMA,
            pltpu.SemaphoreType.DMA,
            pltpu.SemaphoreType.DMA,
        ],
    )
    def body(emb_hbm, idx_hbm, out_hbm, idx_v,
             buf0, buf1, buf2, buf3, sem0, sem1, sem2, sem3):
        bufs = (buf0, buf1, buf2, buf3)
        sems = (sem0, sem1, sem2, sem3)
        wid = lax.axis_index("s") * _NC + lax.axis_index("c")
        pltpu.sync_copy(idx_hbm.at[wid], idx_v)
        cbase = wid * _CH_S
        for u in range(3):
            pltpu.make_async_copy(emb_hbm.at[idx_v.at[u]], bufs[u], sems[u]).start()

        def step(t, carry):
            j0 = t * 4
            for u in range(4):
                j = j0 + u
                nb = bufs[(u + 3) % 4]
                ns = sems[(u + 3) % 4]

                @pl.when(j + 3 < _CH_S)
                def _():
                    pltpu.make_async_copy(emb_hbm.at[idx_v.at[j + 3]], nb, ns).start()

                pltpu.make_async_copy(emb_hbm.at[idx_v.at[j]], bufs[u], sems[u]).wait()
                pltpu.sync_copy(bufs[u], out_hbm.at[cbase + j])
            return carry

        lax.fori_loop(0, _CH_S // 4, step, 0)

    return body(emb, idx3)


_BB = 512


def _tc_body(dense, gath, w0, b0, w1, b1, w2, b2,
             wh, wint, tb0, wt1, tb1, wt2, tb2, wt3, tb3, wt4, tb4, out):
    f32 = jnp.float32
    h = jnp.maximum(jnp.dot(dense[...], w0[...], preferred_element_type=f32) + b0[...], 0.0)
    h = jnp.maximum(jnp.dot(h, w1[...], preferred_element_type=f32) + b1[...], 0.0)
    h2 = jnp.maximum(jnp.dot(h, w2[...], preferred_element_type=f32) + b2[...], 0.0)
    g3 = gath[...].reshape(_BB * _NP, _EMBED).reshape(_BB, _NP, _EMBED)
    slot = lax.broadcasted_iota(jnp.int32, (_BB, _NP, _EMBED), 1)
    comb = jnp.where(slot == _NS, h2.reshape(_BB, 1, _EMBED), g3).astype(jnp.bfloat16)
    inter = lax.dot_general(comb, comb, (((2,), (2,)), ((0,), (0,))),
                            preferred_element_type=f32)
    interf = inter.reshape(_BB, _NP * _NP)
    t = jnp.dot(h2, wh[...], preferred_element_type=f32)
    t = t + jnp.dot(interf, wint[...], preferred_element_type=f32)
    t = jnp.maximum(t + tb0[...], 0.0)
    t = jnp.maximum(jnp.dot(t, wt1[...], preferred_element_type=f32) + tb1[...], 0.0)
    t = jnp.maximum(jnp.dot(t, wt2[...], preferred_element_type=f32) + tb2[...], 0.0)
    t = jnp.maximum(jnp.dot(t, wt3[...], preferred_element_type=f32) + tb3[...], 0.0)
    out[...] = jnp.dot(t, wt4[...], preferred_element_type=f32) + tb4[...]


def _tc_fused(dense, gath3, weights):
    grid = (_BS // _BB,)
    nch = _BB * _NP // _CHUNK

    def blk(shape):
        return pl.BlockSpec(shape, lambda i: (i, 0))

    def rep(shape):
        return pl.BlockSpec(shape, lambda i: (0,) * len(shape))

    in_specs = [blk((_BB, _ND)),
                pl.BlockSpec((nch, _CHUNK, _EMBED), lambda i: (i, 0, 0))]
    for w in weights:
        in_specs.append(rep(w.shape))
    return pl.pallas_call(
        _tc_body,
        grid=grid,
        in_specs=in_specs,
        out_specs=blk((_BB, 1)),
        out_shape=jax.ShapeDtypeStruct((_BS, 1), jnp.float32),
    )(dense, gath3, *weights)


def kernel(x, emb, bw0, bb0, bw1, bb1, bw2, bb2,
           tw0, tb0, tw1, tb1, tw2, tb2, tw3, tb3, tw4, tb4):
    dense = x[:, :_ND]
    idx_all = x[:, _ND:].astype(jnp.int32) % _VOCAB
    npd = _NP - _NS
    dummy = (jnp.arange(_B, dtype=jnp.int32)[:, None] * npd
             + jnp.arange(npd, dtype=jnp.int32)[None, :]) % _VOCAB
    idx_pad = jnp.concatenate([idx_all, dummy], axis=1)

    iu, ku = np.triu_indices(_NS + 1)
    remap = np.concatenate([[26], np.arange(_NS)])
    ri = remap[iu]
    rk = remap[ku]
    lo = np.minimum(ri, rk)
    hi = np.maximum(ri, rk)
    rows = jnp.asarray(lo * _NP + hi, dtype=jnp.int32)
    wint = jnp.zeros((_NP * _NP, 1024), jnp.float32).at[rows].set(tw0[:, _EMBED:].T)

    weights = (
        bw0.T, bb0.reshape(1, -1), bw1.T, bb1.reshape(1, -1), bw2.T, bb2.reshape(1, -1),
        tw0[:, :_EMBED].T, wint, tb0.reshape(1, -1),
        tw1.T, tb1.reshape(1, -1), tw2.T, tb2.reshape(1, -1),
        tw3.T, tb3.reshape(1, -1), tw4.T, tb4.reshape(1, -1),
    )

    parts = []
    for s in range(_NSLICE):
        idx_s = idx_pad[s * _BS:(s + 1) * _BS].reshape(_NW, _CH_S, _CHUNK)
        g = _sc_gather(emb, idx_s)
        parts.append(_tc_fused(dense[s * _BS:(s + 1) * _BS], g, weights))
    return jnp.concatenate(parts, axis=0)
